# baseline (speedup 1.0000x reference)
import functools

import jax
import jax.numpy as jnp
from jax import lax
from jax.experimental import pallas as pl
from jax.experimental.pallas import tpu as pltpu
from jax.experimental.pallas import tpu_sc as plsc


def _gather_call(seq, bt, d, nw, nc, idx4, wtp):
    g = bt // nw
    nbuf = 4
    half = 2
    mesh = plsc.VectorSubcoreMesh(core_axis_name="c", subcore_axis_name="s")
    scratch = [
        pltpu.VMEM((seq, g), jnp.int32),
        pltpu.VMEM((nbuf, g, 2 * d), jnp.float32),
        pltpu.VMEM((half, g, d), jnp.float32),
    ] + [pltpu.SemaphoreType.DMA] * (nbuf + half)

    @functools.partial(
        pl.kernel,
        mesh=mesh,
        out_type=jax.ShapeDtypeStruct((bt, seq, d), jnp.float32),
        compiler_params=pltpu.CompilerParams(use_tc_tiling_on_sc=True),
        scratch_types=scratch,
    )
    def k(idx_hbm, tab_hbm, out_hbm, idx_v, g_v, o_v, *sems):
        gsem = sems[:nbuf]
        osem = sems[nbuf:]
        wid = lax.axis_index("s") * nc + lax.axis_index("c")
        pltpu.sync_copy(idx_hbm.at[wid], idx_v)
        row0 = wid * g

        def gather_desc(si, slot):
            return pltpu.make_async_copy(
                tab_hbm.at[idx_v.at[si]], g_v.at[slot], gsem[slot]
            )

        def out_desc(si, oslot):
            return pltpu.make_async_copy(
                o_v.at[oslot],
                out_hbm.at[pl.ds(row0, g), si, :],
                osem[oslot],
            )

        def compact(slot, oslot):
            gref = g_v.at[slot]
            oref = o_v.at[oslot]

            def body(i, carry):
                for u in range(d // 16):
                    oref[i, pl.ds(16 * u, 16)] = gref[i, pl.ds(16 * u, 16)]
                return carry

            lax.fori_loop(0, g, body, 0)

        for si in range(nbuf):
            gather_desc(si, si).start()

        def blk(bi, carry):
            for bsl in range(nbuf):
                si = bi * nbuf + bsl
                prev = (bsl - 1) % half
                gather_desc(si, bsl).wait()

                @pl.when(si >= 1)
                def _():
                    out_desc(lax.max(si - 1, 0), prev).start()

                @pl.when(si >= half)
                def _():
                    out_desc(lax.max(si - half, 0), bsl % half).wait()

                compact(bsl, bsl % half)

                @pl.when(si + nbuf < seq)
                def _():
                    gather_desc(lax.min(si + nbuf, seq - 1), bsl).start()

            return carry

        lax.fori_loop(0, seq // nbuf, blk, 0)
        out_desc(seq - 1, (seq - 1) % half).start()
        for i in range(half):
            si = seq - half + i
            out_desc(si, si % half).wait()

    return k(idx4, wtp)


def kernel(token_ids, weight):
    bt, seq = token_ids.shape
    v, d = weight.shape
    info = plsc.get_sparse_core_info()
    nc, ns = info.num_cores, info.num_subcores
    nw = nc * ns
    g = bt // nw
    wtp = jnp.concatenate([weight, weight], axis=1)
    idx4 = token_ids.reshape(nw, g, seq).transpose(0, 2, 1)
    return _gather_call(seq, bt, d, nw, nc, idx4, wtp)


# speedup vs baseline: 1.1359x; 1.1359x over previous
import functools

import jax
import jax.numpy as jnp
from jax import lax
from jax.experimental import pallas as pl
from jax.experimental.pallas import tpu as pltpu
from jax.experimental.pallas import tpu_sc as plsc


def _gather_call(seq, bt, d, nw, nc, idx4, wtp):
    g = bt // nw
    nbuf = 4
    half = 2
    mesh = plsc.VectorSubcoreMesh(core_axis_name="c", subcore_axis_name="s")
    scratch = [
        pltpu.VMEM((seq, g), jnp.int32),
        pltpu.VMEM((nbuf, g, 2 * d), jnp.float32),
        pltpu.VMEM((half, g, d), jnp.float32),
    ] + [pltpu.SemaphoreType.DMA] * (nbuf + half)

    @functools.partial(
        pl.kernel,
        mesh=mesh,
        out_type=jax.ShapeDtypeStruct((bt, seq, d), jnp.float32),
        compiler_params=pltpu.CompilerParams(use_tc_tiling_on_sc=True),
        scratch_types=scratch,
    )
    def k(idx_hbm, tab_hbm, out_hbm, idx_v, g_v, o_v, *sems):
        gsem = sems[:nbuf]
        osem = sems[nbuf:]
        wid = lax.axis_index("s") * nc + lax.axis_index("c")
        pltpu.sync_copy(idx_hbm.at[wid], idx_v)
        row0 = wid * g

        def gather_desc(si, slot):
            return pltpu.make_async_copy(
                tab_hbm.at[idx_v.at[si]], g_v.at[slot], gsem[slot]
            )

        def out_desc(si, oslot):
            return pltpu.make_async_copy(
                o_v.at[oslot],
                out_hbm.at[pl.ds(row0, g), si, :],
                osem[oslot],
            )

        def compact(slot, oslot):
            gref = g_v.at[slot]
            oref = o_v.at[oslot]

            def body(i, carry):
                for u in range(d // 16):
                    oref[i, pl.ds(16 * u, 16)] = gref[i, pl.ds(16 * u, 16)]
                return carry

            lax.fori_loop(0, g, body, 0)

        for si in range(nbuf):
            gather_desc(si, si).start()

        def blk(bi, carry):
            for bsl in range(nbuf):
                si = bi * nbuf + bsl
                prev = (bsl - 1) % half
                gather_desc(si, bsl).wait()

                @pl.when(si >= 1)
                def _():
                    out_desc(lax.max(si - 1, 0), prev).start()

                @pl.when(si >= half)
                def _():
                    out_desc(lax.max(si - half, 0), bsl % half).wait()

                compact(bsl, bsl % half)

                @pl.when(si + nbuf < seq)
                def _():
                    gather_desc(lax.min(si + nbuf, seq - 1), bsl).start()

            return carry

        lax.fori_loop(0, seq // nbuf, blk, 0)
        out_desc(seq - 1, (seq - 1) % half).start()
        for i in range(half):
            si = seq - half + i
            out_desc(si, si % half).wait()

    return k(idx4, wtp)


def kernel(token_ids, weight):
    bt, seq = token_ids.shape
    v, d = weight.shape
    info = plsc.get_sparse_core_info()
    nc, ns = info.num_cores, info.num_subcores
    nw = nc * ns
    g = bt // nw
    wtp = jnp.pad(weight, ((0, 0), (0, d)))
    idx4 = token_ids.reshape(nw, g, seq).transpose(0, 2, 1)
    return _gather_call(seq, bt, d, nw, nc, idx4, wtp)
